# baseline (speedup 1.0000x reference)
import functools

import jax
import jax.numpy as jnp
from jax import lax
from jax.experimental import pallas as pl
from jax.experimental.pallas import tpu as pltpu, tpu_sc as plsc

NC = 2
NS = 16
L = 16
M = 128
NPH = 8


def _seft_body(x_hbm, mask_hbm, out_hbm, mask_v, j_v, mrow_v, rowids,
               m_buf, out_buf, sem0, sem1, wsem0, wsem1):
    wid = lax.axis_index("s") * NC + lax.axis_index("c")
    sems = (sem0, sem1)
    wsems = (wsem0, wsem1)

    iota = lax.iota(jnp.int32, L)
    pat = lax.shift_left(lax.shift_right_logical(iota, 3), 7) + (iota & 7)

    def fire_stage(ph):
        pp = ph % 2
        u = wid * 2 + ph // 4
        grp = u // 4
        hp = u % 4
        base_u = hp * 1024 + grp * 8 + (ph % 4) * 256 + pat
        for m in range(16):
            s = m // 8
            rowids[pp * 2 + s, pl.ds((m % 8) * L, L)] = (
                base_u + ((m // 4) * 16 + (m % 4) * 4) * 1024)
        return [
            pltpu.async_copy(
                x_hbm.at[rowids.at[pp * 2 + s]],
                m_buf.at[pl.ds(pp * 256 + s * 128, 128)], sems[pp])
            for s in range(2)
        ]

    stage_cps = {0: fire_stage(0), 1: fire_stage(1)}

    pltpu.sync_copy(mask_hbm, mask_v)
    for c in range(M // L):
        mv = mask_v[pl.ds(c * L, L)]
        idx = plsc.load_gather(mask_v, [mv])
        j_v[pl.ds(c * L, L)] = lax.shift_right_logical(idx, 4)
        mrow_v[pl.ds(c * L, L)] = lax.shift_left(idx & 15, 4)

    j_k = [j_v[pl.ds(k * L, L)] for k in range(8)]
    mrow_k = [mrow_v[pl.ds(k * L, L)] for k in range(8)]

    write_cps = {}
    for ph in range(NPH):
        pp = ph % 2
        for cp in write_cps.pop(pp, []):
            cp.wait()
        for cp in stage_cps.pop(ph):
            cp.wait()

        mofs = pp * 256
        oofs = pp * 128

        def compute(it, carry):
            for wp in (it * 2, it * 2 + 1):
                rv = [mrow_k[k] + (mofs + wp) for k in range(8)]
                for b in range(8):
                    row = oofs + b * L + wp
                    gs = [plsc.load_gather(m_buf, [rv[k], j_k[k] + b * L])
                          for k in range(8)]
                    for k in range(8):
                        out_buf[row, pl.ds(k * L, L)] = gs[k]
            return carry

        lax.fori_loop(0, L // 2, compute, 0)

        if ph + 2 < NPH:
            stage_cps[ph + 2] = fire_stage(ph + 2)

        u = wid * 2 + ph // 4
        obase = (u // 4) * 2048 + (u % 4) * 64 + (ph % 4) * L
        write_cps[pp] = [
            pltpu.async_copy(
                out_buf.at[pl.ds(oofs + b * L, L)],
                out_hbm.at[pl.ds(obase + b * 256, L)], wsems[pp])
            for b in range(8)
        ]
    for cps in write_cps.values():
        for cp in cps:
            cp.wait()


@jax.jit
def kernel(x, mask):
    xe = x.transpose(1, 2, 3, 0)
    xe = xe.reshape(4, 16, 8, 8, 16, 128)
    xe = xe.transpose(0, 1, 2, 4, 3, 5)
    xe = xe.reshape(65536, 128)

    mesh = plsc.VectorSubcoreMesh(core_axis_name="c", subcore_axis_name="s")
    k = functools.partial(
        pl.kernel,
        mesh=mesh,
        compiler_params=pltpu.CompilerParams(needs_layout_passes=False),
        out_type=jax.ShapeDtypeStruct((32768, 128), jnp.float32),
        scratch_types=[
            pltpu.VMEM((M,), jnp.int32),
            pltpu.VMEM((M,), jnp.int32),
            pltpu.VMEM((M,), jnp.int32),
            pltpu.VMEM((4, M), jnp.int32),
            pltpu.VMEM((512, 128), jnp.float32),
            pltpu.VMEM((256, 128), jnp.float32),
            pltpu.SemaphoreType.DMA,
            pltpu.SemaphoreType.DMA,
            pltpu.SemaphoreType.DMA,
            pltpu.SemaphoreType.DMA,
        ],
    )(_seft_body)
    out = k(xe, mask)
    return out.reshape(128, 4, 16, 4, 128).transpose(0, 4, 1, 2, 3)


# speedup vs baseline: 1.0843x; 1.0843x over previous
import functools

import jax
import jax.numpy as jnp
from jax import lax
from jax.experimental import pallas as pl
from jax.experimental.pallas import tpu as pltpu, tpu_sc as plsc

NC = 2
NS = 16
L = 16
M = 128
NPH = 8


def _seft_body(x_hbm, mask_hbm, out_hbm, mask_v, j_v, mrow_v, rowids,
               m_buf, out_buf, sem0, sem1, wsem0, wsem1):
    wid = lax.axis_index("s") * NC + lax.axis_index("c")
    sems = (sem0, sem1)
    wsems = (wsem0, wsem1)

    iota = lax.iota(jnp.int32, L)
    pat = lax.shift_left(lax.shift_right_logical(iota, 3), 7) + (iota & 7)

    def fire_stage(ph):
        pp = ph % 2
        u = wid * 2 + ph // 4
        grp = u // 4
        hp = u % 4
        base_u = hp * 1024 + grp * 8 + (ph % 4) * 256 + pat
        for m in range(16):
            s = m // 8
            rowids[pp * 2 + s, pl.ds((m % 8) * L, L)] = (
                base_u + ((m // 4) * 16 + (m % 4) * 4) * 1024)
        return [
            pltpu.async_copy(
                x_hbm.at[rowids.at[pp * 2 + s]],
                m_buf.at[pl.ds(pp * 256 + s * 128, 128)], sems[pp])
            for s in range(2)
        ]

    stage_cps = {0: fire_stage(0), 1: fire_stage(1)}

    pltpu.sync_copy(mask_hbm, mask_v)
    for c in range(M // L):
        mv = mask_v[pl.ds(c * L, L)]
        idx = plsc.load_gather(mask_v, [mv])
        j_v[pl.ds(c * L, L)] = lax.shift_right_logical(idx, 4)
        mrow_v[pl.ds(c * L, L)] = lax.shift_left(idx & 15, 4)

    j_k = [j_v[pl.ds(k * L, L)] for k in range(8)]
    mrow_k = [mrow_v[pl.ds(k * L, L)] for k in range(8)]

    write_cps = {}
    for ph in range(NPH):
        pp = ph % 2
        for cp in write_cps.pop(pp, []):
            cp.wait()
        for cp in stage_cps.pop(ph):
            cp.wait()

        mofs = pp * 256

        def compute(wp, carry):
            rv = [mrow_k[k] + (mofs + wp) for k in range(8)]
            for b in range(8):
                gs = [plsc.load_gather(m_buf, [rv[k], j_k[k] + b * L])
                      for k in range(8)]
                for k in range(8):
                    out_buf[pp, b, wp, pl.ds(k * L, L)] = gs[k]
            return carry

        lax.fori_loop(0, L, compute, 0)

        if ph + 2 < NPH:
            stage_cps[ph + 2] = fire_stage(ph + 2)

        u = wid * 2 + ph // 4
        grp = u // 4
        e0 = (u % 4) * 64 + (ph % 4) * L
        write_cps[pp] = [
            pltpu.async_copy(
                out_buf.at[pp],
                out_hbm.at[pl.ds(grp * 8, 8), pl.ds(e0, L)], wsems[pp])
        ]
    for cps in write_cps.values():
        for cp in cps:
            cp.wait()


@jax.jit
def kernel(x, mask):
    xe = x.transpose(1, 2, 3, 0)
    xe = xe.reshape(4, 16, 8, 8, 16, 128)
    xe = xe.transpose(0, 1, 2, 4, 3, 5)
    xe = xe.reshape(65536, 128)

    mesh = plsc.VectorSubcoreMesh(core_axis_name="c", subcore_axis_name="s")
    k = functools.partial(
        pl.kernel,
        mesh=mesh,
        compiler_params=pltpu.CompilerParams(needs_layout_passes=False),
        out_type=jax.ShapeDtypeStruct((128, 256, 128), jnp.float32),
        scratch_types=[
            pltpu.VMEM((M,), jnp.int32),
            pltpu.VMEM((M,), jnp.int32),
            pltpu.VMEM((M,), jnp.int32),
            pltpu.VMEM((4, M), jnp.int32),
            pltpu.VMEM((512, 128), jnp.float32),
            pltpu.VMEM((2, 8, L, 128), jnp.float32),
            pltpu.SemaphoreType.DMA,
            pltpu.SemaphoreType.DMA,
            pltpu.SemaphoreType.DMA,
            pltpu.SemaphoreType.DMA,
        ],
    )(_seft_body)
    out = k(xe, mask)
    return out.reshape(128, 4, 16, 4, 128).transpose(0, 4, 1, 2, 3)
